# Initial kernel scaffold; baseline (speedup 1.0000x reference)
import jax
import jax.numpy as jnp
from jax.experimental import pallas as pl


def kernel(x, x1, edge_index, edge_index1, W11, b11, W21, b21, Wl12, Wr12, b12, Wl22, Wr22, b22, mlp_w1, mlp_b1, ln_g, ln_b, mlp_w2, mlp_b2):
    raise NotImplementedError("write your pallas kernel here")


# speedup vs baseline: 11.1452x; 11.1452x over previous
import functools

import jax
import jax.numpy as jnp
from jax import lax
from jax.experimental import pallas as pl
from jax.experimental.pallas import tpu as pltpu
from jax.experimental.pallas import tpu_sc as plsc

N = 10000
E = 320000
D = 128
H = 128
HH = 64

NC = 2
NS = 16
NPAD = 10240
RPT = NPAD // NS
K = 80
EPT = E // NS
NCH = EPT // K

_MESH = plsc.VectorSubcoreMesh(
    core_axis_name="c", subcore_axis_name="s", num_cores=NC, num_subcores=NS)


def _make_sc_scatter(W, gather):
  scratch = [
      pltpu.VMEM((K,), jnp.int32),
      pltpu.VMEM((K,), jnp.int32),
      pltpu.VMEM((K, W), jnp.float32),
      pltpu.VMEM_SHARED((NPAD, W), jnp.float32),
      pltpu.SemaphoreType.DMA,
  ]
  out_type = (jax.ShapeDtypeStruct((NPAD, W), jnp.float32),
              jax.ShapeDtypeStruct((NPAD, W), jnp.float32))

  def body(*refs):
    if gather:
      (src1, dst1, tab1, src2, dst2, tab2, zeros,
       out1, out2, sidx, didx, rows, acc, sem) = refs
    else:
      (dst1, dst2, ones, zeros,
       out1, out2, sidx, didx, rows, acc, sem) = refs
    c = lax.axis_index("c")
    s = lax.axis_index("s")
    r0 = s * RPT
    pltpu.sync_copy(zeros.at[pl.ds(r0, RPT)], acc.at[pl.ds(r0, RPT)])
    if not gather:
      pltpu.sync_copy(ones, rows)
    plsc.subcore_barrier()

    def chunk(j, carry):
      base = s * EPT + j * K

      @pl.when(c == 0)
      def _():
        pltpu.sync_copy(dst1.at[pl.ds(base, K)], didx)
        if gather:
          pltpu.sync_copy(src1.at[pl.ds(base, K)], sidx)
          pltpu.async_copy(tab1.at[sidx], rows, sem).wait()

      @pl.when(c == 1)
      def _():
        pltpu.sync_copy(dst2.at[pl.ds(base, K)], didx)
        if gather:
          pltpu.sync_copy(src2.at[pl.ds(base, K)], sidx)
          pltpu.async_copy(tab2.at[sidx], rows, sem).wait()

      pltpu.sync_copy(rows, acc.at[didx], add=True)
      return carry

    lax.fori_loop(0, NCH, chunk, 0)
    plsc.subcore_barrier()

    @pl.when(c == 0)
    def _():
      pltpu.sync_copy(acc.at[pl.ds(r0, RPT)], out1.at[pl.ds(r0, RPT)])

    @pl.when(c == 1)
    def _():
      pltpu.sync_copy(acc.at[pl.ds(r0, RPT)], out2.at[pl.ds(r0, RPT)])

  return pl.kernel(body, out_type=out_type, mesh=_MESH, scratch_types=scratch,
                   compiler_params=pltpu.CompilerParams(use_tc_tiling_on_sc=False))


_sc_hist = _make_sc_scatter(16, gather=False)
_sc_prop128 = _make_sc_scatter(H, gather=True)
_sc_prop64 = _make_sc_scatter(HH, gather=True)

R = 1000
G = N // R


def _tc_a_body(x_ref, x1_ref, w11_ref, w21_ref, c1_ref, c2_ref,
               h1p_ref, h2p_ref):
  dis1 = lax.rsqrt(c1_ref[:, 0:1] + 1.0)
  dis2 = lax.rsqrt(c2_ref[:, 0:1] + 1.0)
  h1 = jnp.dot(x_ref[...], w11_ref[...], preferred_element_type=jnp.float32)
  h2 = jnp.dot(x1_ref[...], w21_ref[...], preferred_element_type=jnp.float32)
  h1p_ref[...] = h1 * dis1
  h2p_ref[...] = h2 * dis2


def _tc_a(x, x1, W11, W21, cnt1, cnt2):
  rb = lambda i: (i, 0)
  z = lambda i: (0, 0)
  return pl.pallas_call(
      _tc_a_body,
      grid=(G,),
      in_specs=[
          pl.BlockSpec((R, D), rb), pl.BlockSpec((R, D), rb),
          pl.BlockSpec((D, H), z), pl.BlockSpec((D, H), z),
          pl.BlockSpec((R, 16), rb), pl.BlockSpec((R, 16), rb),
      ],
      out_specs=[pl.BlockSpec((R, H), rb), pl.BlockSpec((R, H), rb)],
      out_shape=[jax.ShapeDtypeStruct((N, H), jnp.float32),
                 jax.ShapeDtypeStruct((N, H), jnp.float32)],
  )(x, x1, W11, W21, cnt1, cnt2)


def _tc_b_body(p1_ref, p2_ref, h1p_ref, h2p_ref, c1_ref, c2_ref,
               b11_ref, b21_ref, wl1_ref, wl2_ref,
               x0_ref, x1b_ref, yl1_ref, yl2_ref):
  dis1 = lax.rsqrt(c1_ref[:, 0:1] + 1.0)
  dis2 = lax.rsqrt(c2_ref[:, 0:1] + 1.0)
  xa = dis1 * (p1_ref[...] + h1p_ref[...]) + b11_ref[...]
  xb = dis2 * (p2_ref[...] + h2p_ref[...]) + b21_ref[...]
  x0a = jnp.maximum(xa, 0.0)
  x1_0 = jnp.maximum(xb, 0.0)
  x0 = x0a + x1_0
  x1b = x1_0 + xb
  x0_ref[...] = x0
  x1b_ref[...] = x1b
  yl1_ref[...] = jnp.dot(x0, wl1_ref[...], preferred_element_type=jnp.float32)
  yl2_ref[...] = jnp.dot(x1b, wl2_ref[...], preferred_element_type=jnp.float32)


def _tc_b(P1, P2, h1p, h2p, cnt1, cnt2, b11, b21, Wl12, Wl22):
  rb = lambda i: (i, 0)
  z = lambda i: (0, 0)
  return pl.pallas_call(
      _tc_b_body,
      grid=(G,),
      in_specs=[
          pl.BlockSpec((R, H), rb), pl.BlockSpec((R, H), rb),
          pl.BlockSpec((R, H), rb), pl.BlockSpec((R, H), rb),
          pl.BlockSpec((R, 16), rb), pl.BlockSpec((R, 16), rb),
          pl.BlockSpec((1, H), z), pl.BlockSpec((1, H), z),
          pl.BlockSpec((H, HH), z), pl.BlockSpec((H, HH), z),
      ],
      out_specs=[pl.BlockSpec((R, H), rb), pl.BlockSpec((R, H), rb),
                 pl.BlockSpec((R, HH), rb), pl.BlockSpec((R, HH), rb)],
      out_shape=[jax.ShapeDtypeStruct((N, H), jnp.float32),
                 jax.ShapeDtypeStruct((N, H), jnp.float32),
                 jax.ShapeDtypeStruct((N, HH), jnp.float32),
                 jax.ShapeDtypeStruct((N, HH), jnp.float32)],
  )(P1, P2, h1p, h2p, cnt1, cnt2, b11, b21, Wl12, Wl22)


def _mlp_rows(xin, w1, b1, g, beta, w2t, b2):
  hm = jnp.dot(xin, w1, preferred_element_type=jnp.float32) + b1
  mu = jnp.mean(hm, axis=-1, keepdims=True)
  var = jnp.mean((hm - mu) ** 2, axis=-1, keepdims=True)
  ln = (hm - mu) * lax.rsqrt(var + 1e-5) * g + beta
  return jnp.sum(ln * w2t, axis=-1, keepdims=True) + b2


def _tc_c_body(q1_ref, q2_ref, c1_ref, c2_ref, x0_ref, x1b_ref,
               wr1_ref, wr2_ref, b12_ref, b22_ref,
               w1_ref, mb1_ref, g_ref, beta_ref, w2t_ref, b2_ref,
               out_ref):
  i = pl.program_id(0)
  inv1 = 1.0 / jnp.maximum(c1_ref[:, 0:1], 1.0)
  inv2 = 1.0 / jnp.maximum(c2_ref[:, 0:1], 1.0)
  xc = jnp.maximum(
      q1_ref[...] * inv1
      + jnp.dot(x0_ref[...], wr1_ref[...], preferred_element_type=jnp.float32)
      + b12_ref[...], 0.0)
  xd = jnp.maximum(
      q2_ref[...] * inv2
      + jnp.dot(x1b_ref[...], wr2_ref[...], preferred_element_type=jnp.float32)
      + b22_ref[...], 0.0)
  w1 = w1_ref[...]
  mb1 = mb1_ref[...]
  g = g_ref[...]
  beta = beta_ref[...]
  w2t = w2t_ref[...]
  b2 = b2_ref[...]
  ra = _mlp_rows(xc, w1, mb1, g, beta, w2t, b2)
  rb = _mlp_rows(xd, w1, mb1, g, beta, w2t, b2)
  partial = (jnp.sum(ra) + jnp.sum(rb)).reshape(1, 1)

  @pl.when(i == 0)
  def _():
    out_ref[...] = jnp.zeros((1, 1), jnp.float32)

  out_ref[...] += partial

  @pl.when(i == G - 1)
  def _():
    out_ref[...] = out_ref[...] * (1.0 / (2.0 * N))


def _tc_c(Q1, Q2, cnt1, cnt2, x0, x1b, Wr12, Wr22, b12, b22,
          w1, mb1, g, beta, w2t, b2):
  rb_ = lambda i: (i, 0)
  z = lambda i: (0, 0)
  return pl.pallas_call(
      _tc_c_body,
      grid=(G,),
      in_specs=[
          pl.BlockSpec((R, HH), rb_), pl.BlockSpec((R, HH), rb_),
          pl.BlockSpec((R, 16), rb_), pl.BlockSpec((R, 16), rb_),
          pl.BlockSpec((R, H), rb_), pl.BlockSpec((R, H), rb_),
          pl.BlockSpec((H, HH), z), pl.BlockSpec((H, HH), z),
          pl.BlockSpec((1, HH), z), pl.BlockSpec((1, HH), z),
          pl.BlockSpec((HH, HH), z), pl.BlockSpec((1, HH), z),
          pl.BlockSpec((1, HH), z), pl.BlockSpec((1, HH), z),
          pl.BlockSpec((1, HH), z), pl.BlockSpec((1, 1), z),
      ],
      out_specs=pl.BlockSpec((1, 1), z),
      out_shape=jax.ShapeDtypeStruct((1, 1), jnp.float32),
  )(Q1, Q2, cnt1, cnt2, x0, x1b, Wr12, Wr22, b12, b22,
    w1, mb1, g, beta, w2t, b2)


def kernel(x, x1, edge_index, edge_index1, W11, b11, W21, b21,
           Wl12, Wr12, b12, Wl22, Wr22, b22,
           mlp_w1, mlp_b1, ln_g, ln_b, mlp_w2, mlp_b2):
  ei = edge_index.astype(jnp.int32)
  ei1 = edge_index1.astype(jnp.int32)
  src1, dst1 = ei[0], ei[1]
  src2, dst2 = ei1[0], ei1[1]

  ones16 = jnp.ones((K, 16), jnp.float32)
  z16 = jnp.zeros((NPAD, 16), jnp.float32)
  z128 = jnp.zeros((NPAD, H), jnp.float32)
  z64 = jnp.zeros((NPAD, HH), jnp.float32)

  cnt1, cnt2 = _sc_hist(dst1, dst2, ones16, z16)
  h1p, h2p = _tc_a(x, x1, W11, W21, cnt1, cnt2)
  P1, P2 = _sc_prop128(src1, dst1, h1p, src2, dst2, h2p, z128)
  x0, x1b, yl1, yl2 = _tc_b(P1, P2, h1p, h2p, cnt1, cnt2,
                            b11.reshape(1, H), b21.reshape(1, H), Wl12, Wl22)
  Q1, Q2 = _sc_prop64(src1, dst1, yl1, src2, dst2, yl2, z64)
  out = _tc_c(Q1, Q2, cnt1, cnt2, x0, x1b, Wr12, Wr22,
              b12.reshape(1, HH), b22.reshape(1, HH),
              mlp_w1, mlp_b1.reshape(1, HH), ln_g.reshape(1, HH),
              ln_b.reshape(1, HH), mlp_w2.reshape(1, HH),
              mlp_b2.reshape(1, 1))
  return out
